# Initial kernel scaffold; baseline (speedup 1.0000x reference)
import jax
import jax.numpy as jnp
from jax.experimental import pallas as pl


def kernel(x1, g, w, w1, b1, gamma, beta):
    raise NotImplementedError("write your pallas kernel here")


# speedup vs baseline: 1.6020x; 1.6020x over previous
import functools

import jax
import jax.numpy as jnp
from jax.experimental import pallas as pl
from jax.experimental.pallas import tpu as pltpu

_EPS = 1e-5
_VMEM_LIMIT = 48 * 1024 * 1024


def _proj_mix_kernel(x_ref, g_ref, wcat_ref, s_ref, m_ref, z_ref, st_ref):
    cout = z_ref.shape[1]
    x = x_ref[0]
    p = jnp.dot(wcat_ref[...], x, preferred_element_type=jnp.float32)
    a = p[:cout]
    c = p[cout:]
    gb = jnp.dot(s_ref[...], g_ref[0],
                 preferred_element_type=jnp.float32) * m_ref[...]
    z = jnp.dot(a, gb, preferred_element_type=jnp.float32) + c
    z_ref[0] = z.astype(z_ref.dtype)
    s1 = jnp.sum(z, axis=1, keepdims=True)
    s2 = jnp.sum(z * z, axis=1, keepdims=True)
    st_ref[0] = jnp.concatenate([s1, s2], axis=1).astype(st_ref.dtype)


def _bn_relu_kernel(n_total, z_ref, st_ref, ga_ref, be_ref, y_ref):
    tot = jnp.sum(st_ref[...].astype(jnp.float32), axis=0)
    inv_n = 1.0 / n_total
    mean = tot[:, 0:1] * inv_n
    var = jnp.maximum(tot[:, 1:2] * inv_n - mean * mean, 0.0)
    inv = jax.lax.rsqrt(var + _EPS)
    scale = inv * ga_ref[...].astype(jnp.float32)
    shift = be_ref[...].astype(jnp.float32) - mean * scale
    z = z_ref[0].astype(jnp.float32)
    y_ref[0] = jnp.maximum(z * scale + shift, 0.0).astype(y_ref.dtype)


@jax.jit
def _forward(x1, g, w, w1, gamma, beta):
    B, Cin, J, T = x1.shape
    Cout = w.shape[0]
    JT = J * T

    x2 = x1.reshape(B, Cin, JT)
    gflat = jnp.transpose(g, (0, 3, 2, 1)).reshape(B, J, JT)
    wcat = jnp.concatenate([w, w1], axis=0)

    r = jnp.arange(JT, dtype=jnp.int32)
    s_sel = (r[:, None] // T == jnp.arange(J, dtype=jnp.int32)[None, :])
    s_sel = s_sel.astype(x1.dtype)
    eqmask = (r[:, None] % T == r[None, :] % T).astype(x1.dtype)

    z, stats = pl.pallas_call(
        _proj_mix_kernel,
        out_shape=(
            jax.ShapeDtypeStruct((B, Cout, JT), jnp.float32),
            jax.ShapeDtypeStruct((B, Cout, 2), jnp.float32),
        ),
        grid=(B,),
        in_specs=[
            pl.BlockSpec((1, Cin, JT), lambda b: (b, 0, 0)),
            pl.BlockSpec((1, J, JT), lambda b: (b, 0, 0)),
            pl.BlockSpec((2 * Cout, Cin), lambda b: (0, 0)),
            pl.BlockSpec((JT, J), lambda b: (0, 0)),
            pl.BlockSpec((JT, JT), lambda b: (0, 0)),
        ],
        out_specs=(
            pl.BlockSpec((1, Cout, JT), lambda b: (b, 0, 0)),
            pl.BlockSpec((1, Cout, 2), lambda b: (b, 0, 0)),
        ),
        compiler_params=pltpu.CompilerParams(
            dimension_semantics=("parallel",),
            vmem_limit_bytes=_VMEM_LIMIT,
        ),
    )(x2, gflat, wcat, s_sel, eqmask)

    y = pl.pallas_call(
        functools.partial(_bn_relu_kernel, float(B * JT)),
        out_shape=jax.ShapeDtypeStruct((B, Cout, JT), x1.dtype),
        grid=(B,),
        in_specs=[
            pl.BlockSpec((1, Cout, JT), lambda b: (b, 0, 0)),
            pl.BlockSpec((B, Cout, 2), lambda b: (0, 0, 0)),
            pl.BlockSpec((Cout, 1), lambda b: (0, 0)),
            pl.BlockSpec((Cout, 1), lambda b: (0, 0)),
        ],
        out_specs=pl.BlockSpec((1, Cout, JT), lambda b: (b, 0, 0)),
        compiler_params=pltpu.CompilerParams(
            dimension_semantics=("parallel",),
            vmem_limit_bytes=_VMEM_LIMIT,
        ),
    )(z, stats, gamma.reshape(Cout, 1), beta.reshape(Cout, 1))

    return y.reshape(B, Cout, J, T)


def kernel(x1, g, w, w1, b1, gamma, beta):
    del b1
    return _forward(x1, g, w, w1, gamma, beta)
